# baseline (speedup 1.0000x reference)
import functools

import jax
import jax.numpy as jnp
from jax import lax
from jax.experimental import pallas as pl
from jax.experimental.pallas import tpu as pltpu
from jax.experimental.pallas import tpu_sc as plsc

_NUM_ROWS = 64
_ROW_STRIDE = 1024
_D = 512
_NUM_CORES = 2


@functools.partial(
    pl.kernel,
    mesh=plsc.VectorSubcoreMesh(core_axis_name="c", subcore_axis_name="s"),
    out_type=jax.ShapeDtypeStruct((_NUM_ROWS, _D), jnp.float32),
    scratch_types=[
        pltpu.VMEM((_NUM_ROWS,), jnp.int32),
        pltpu.VMEM((_NUM_ROWS, _D), jnp.float32),
        pltpu.SemaphoreType.DMA,
    ],
)
def _gather_rows(x_hbm, out_hbm, idx_v, rows_v, sem):
    wid = lax.axis_index("s") * _NUM_CORES + lax.axis_index("c")

    @pl.when(wid == 0)
    def _():
        for k in range(_NUM_ROWS // 16):
            idx_v[pl.ds(k * 16, 16)] = (
                lax.iota(jnp.int32, 16) + k * 16
            ) * _ROW_STRIDE
        pltpu.async_copy(x_hbm.at[idx_v], rows_v, sem).wait()
        pltpu.sync_copy(rows_v, out_hbm)


def kernel(x):
    return _gather_rows(x)


# speedup vs baseline: 1.1983x; 1.1983x over previous
import functools

import jax
import jax.numpy as jnp
from jax import lax
from jax.experimental import pallas as pl
from jax.experimental.pallas import tpu as pltpu
from jax.experimental.pallas import tpu_sc as plsc

_NUM_ROWS = 64
_ROW_STRIDE = 1024
_D = 512
_NUM_CORES = 1
_NUM_SUBCORES = 16
_NUM_WORKERS = _NUM_CORES * _NUM_SUBCORES
_ROWS_PER_WORKER = _NUM_ROWS // _NUM_WORKERS


@functools.partial(
    pl.kernel,
    mesh=plsc.VectorSubcoreMesh(core_axis_name="c", subcore_axis_name="s", num_cores=1),
    out_type=jax.ShapeDtypeStruct((_NUM_ROWS, _D), jnp.float32),
    scratch_types=[
        pltpu.VMEM((_ROWS_PER_WORKER, _D), jnp.float32),
        pltpu.SemaphoreType.DMA,
    ],
)
def _gather_rows(x_hbm, out_hbm, buf, sem):
    wid = lax.axis_index("s") * _NUM_CORES + lax.axis_index("c")
    base = wid * _ROWS_PER_WORKER
    copies = [
        pltpu.make_async_copy(
            x_hbm.at[pl.ds((base + j) * _ROW_STRIDE, 1)],
            buf.at[pl.ds(j, 1)],
            sem,
        )
        for j in range(_ROWS_PER_WORKER)
    ]
    for c in copies:
        c.start()
    for c in copies:
        c.wait()
    pltpu.sync_copy(buf, out_hbm.at[pl.ds(base, _ROWS_PER_WORKER)])


def kernel(x):
    return _gather_rows(x)


# speedup vs baseline: 1.2559x; 1.0480x over previous
import functools

import jax
import jax.numpy as jnp
from jax import lax
from jax.experimental import pallas as pl
from jax.experimental.pallas import tpu as pltpu
from jax.experimental.pallas import tpu_sc as plsc

_NUM_ROWS = 64
_ROW_STRIDE = 1024
_D = 512


@functools.partial(
    pl.kernel,
    mesh=plsc.ScalarSubcoreMesh(axis_name="c", num_cores=1),
    out_type=jax.ShapeDtypeStruct((_NUM_ROWS, _D), jnp.float32),
    scratch_types=[
        pltpu.VMEM_SHARED((_NUM_ROWS, _D), jnp.float32),
        pltpu.SemaphoreType.DMA,
    ],
)
def _gather_rows(x_hbm, out_hbm, buf, sem):
    copies = [
        pltpu.make_async_copy(
            x_hbm.at[pl.ds(j * _ROW_STRIDE, 1)],
            buf.at[pl.ds(j, 1)],
            sem,
        )
        for j in range(_NUM_ROWS)
    ]
    for c in copies:
        c.start()
    for c in copies:
        c.wait()
    pltpu.sync_copy(buf, out_hbm)


def kernel(x):
    return _gather_rows(x)


# speedup vs baseline: 1.2598x; 1.0031x over previous
import functools

import jax
import jax.numpy as jnp
from jax import lax
from jax.experimental import pallas as pl
from jax.experimental.pallas import tpu as pltpu
from jax.experimental.pallas import tpu_sc as plsc

_NUM_ROWS = 64
_ROW_STRIDE = 1024
_D = 512


@functools.partial(
    pl.kernel,
    mesh=plsc.ScalarSubcoreMesh(axis_name="c", num_cores=1),
    out_type=jax.ShapeDtypeStruct((_NUM_ROWS, _D), jnp.float32),
    scratch_types=[
        pltpu.VMEM_SHARED((_NUM_ROWS, _D), jnp.float32),
        pltpu.SemaphoreType.DMA,
    ],
)
def _gather_rows(x_hbm, out_hbm, buf, sem):
    copies = [
        pltpu.make_async_copy(
            x_hbm.at[pl.ds(j * _ROW_STRIDE, 1)],
            buf.at[pl.ds(j, 1)],
            sem,
        )
        for j in range(_NUM_ROWS)
    ]
    for c in copies:
        c.start()
    pltpu.make_async_copy(x_hbm.at[pl.ds(0, _NUM_ROWS)], buf, sem).wait()
    pltpu.sync_copy(buf, out_hbm)


def kernel(x):
    return _gather_rows(x)


# speedup vs baseline: 1.2627x; 1.0023x over previous
import functools

import jax
import jax.numpy as jnp
from jax import lax
from jax.experimental import pallas as pl
from jax.experimental.pallas import tpu as pltpu
from jax.experimental.pallas import tpu_sc as plsc

_NUM_ROWS = 64
_ROW_STRIDE = 1024
_D = 512


@functools.partial(
    pl.kernel,
    mesh=plsc.ScalarSubcoreMesh(axis_name="c", num_cores=1),
    out_type=jax.ShapeDtypeStruct((_NUM_ROWS, _D), jnp.float32),
    scratch_types=[
        pltpu.VMEM_SHARED((_NUM_ROWS, _D), jnp.float32),
        pltpu.SemaphoreType.DMA,
    ],
)
def _gather_rows(x_hbm, out_hbm, buf, sem):
    def issue_chunk(i, carry):
        base = i * 8
        for j in range(8):
            pltpu.make_async_copy(
                x_hbm.at[pl.ds((base + j) * _ROW_STRIDE, 1)],
                buf.at[pl.ds(base + j, 1)],
                sem,
            ).start()
        return carry

    lax.fori_loop(0, _NUM_ROWS // 8, issue_chunk, 0)
    pltpu.make_async_copy(x_hbm.at[pl.ds(0, _NUM_ROWS)], buf, sem).wait()
    pltpu.sync_copy(buf, out_hbm)


def kernel(x):
    return _gather_rows(x)


# speedup vs baseline: 1.2654x; 1.0022x over previous
import functools

import jax
import jax.numpy as jnp
from jax import lax
from jax.experimental import pallas as pl
from jax.experimental.pallas import tpu as pltpu
from jax.experimental.pallas import tpu_sc as plsc

_NUM_ROWS = 64
_ROW_STRIDE = 1024
_D = 512
_CHUNK = 8
_NUM_CHUNKS = _NUM_ROWS // _CHUNK


@functools.partial(
    pl.kernel,
    mesh=plsc.ScalarSubcoreMesh(axis_name="c", num_cores=1),
    out_type=jax.ShapeDtypeStruct((_NUM_ROWS, _D), jnp.float32),
    scratch_types=[
        pltpu.VMEM_SHARED((_NUM_ROWS, _D), jnp.float32),
    ]
    + [pltpu.SemaphoreType.DMA] * _NUM_CHUNKS
    + [pltpu.SemaphoreType.DMA],
)
def _gather_rows(x_hbm, out_hbm, buf, *sems):
    chunk_sems = sems[:_NUM_CHUNKS]
    store_sem = sems[_NUM_CHUNKS]
    for c in range(_NUM_CHUNKS):
        base = c * _CHUNK
        for j in range(_CHUNK):
            pltpu.make_async_copy(
                x_hbm.at[pl.ds((base + j) * _ROW_STRIDE, 1)],
                buf.at[pl.ds(base + j, 1)],
                chunk_sems[c],
            ).start()
    for c in range(_NUM_CHUNKS):
        base = c * _CHUNK
        pltpu.make_async_copy(
            x_hbm.at[pl.ds(0, _CHUNK)],
            buf.at[pl.ds(base, _CHUNK)],
            chunk_sems[c],
        ).wait()
        pltpu.make_async_copy(
            buf.at[pl.ds(base, _CHUNK)],
            out_hbm.at[pl.ds(base, _CHUNK)],
            store_sem,
        ).start()
    pltpu.make_async_copy(x_hbm.at[pl.ds(0, _NUM_ROWS)], out_hbm, store_sem).wait()


def kernel(x):
    return _gather_rows(x)
